# Initial kernel scaffold; baseline (speedup 1.0000x reference)
import jax
import jax.numpy as jnp
from jax.experimental import pallas as pl


def kernel(x, W):
    raise NotImplementedError("write your pallas kernel here")


# speedup vs baseline: 1.1099x; 1.1099x over previous
import jax
import jax.numpy as jnp
from jax.experimental import pallas as pl

D_MODEL = 4096
NUM_EXPERTS = 64
TOP_K = 8
TOP_C = 16

_NEG_BIG = -3.0e38


def _gating_body(x_ref, wt_ref, idx8_ref, w8_ref, probs_ref, idx16_ref):
    logits = jnp.dot(x_ref[...], wt_ref[...],
                     preferred_element_type=jnp.float32)

    vmax = jnp.max(logits, axis=1, keepdims=True)
    e = jnp.exp(logits - vmax)
    s = jnp.sum(e, axis=1, keepdims=True)
    probs_ref[...] = e / (s + 1e-12)

    iota = jax.lax.broadcasted_iota(jnp.int32, logits.shape, 1)
    vals = logits
    tv, ti = [], []
    for _ in range(TOP_C):
        m = jnp.max(vals, axis=1, keepdims=True)
        idx = jnp.min(jnp.where(vals == m, iota, NUM_EXPERTS),
                      axis=1, keepdims=True)
        tv.append(m)
        ti.append(idx)
        vals = jnp.where(iota == idx, _NEG_BIG, vals)

    idx16_ref[...] = jnp.concatenate(ti, axis=1)
    idx8_ref[...] = jnp.concatenate(ti[:TOP_K], axis=1)

    topv = jnp.concatenate(tv[:TOP_K], axis=1)
    e8 = jnp.exp(topv - tv[0])
    w8_ref[...] = e8 / (jnp.sum(e8, axis=1, keepdims=True) + 1e-12)


def _run(x, W, block_m=512, interpret=False):
    n_tokens = x.shape[0]
    wt = W.T
    grid = (n_tokens // block_m,)
    out = pl.pallas_call(
        _gating_body,
        grid=grid,
        in_specs=[
            pl.BlockSpec((block_m, D_MODEL), lambda i: (i, 0)),
            pl.BlockSpec((D_MODEL, NUM_EXPERTS), lambda i: (0, 0)),
        ],
        out_specs=[
            pl.BlockSpec((block_m, TOP_K), lambda i: (i, 0)),
            pl.BlockSpec((block_m, TOP_K), lambda i: (i, 0)),
            pl.BlockSpec((block_m, NUM_EXPERTS), lambda i: (i, 0)),
            pl.BlockSpec((block_m, TOP_C), lambda i: (i, 0)),
        ],
        out_shape=[
            jax.ShapeDtypeStruct((n_tokens, TOP_K), jnp.int32),
            jax.ShapeDtypeStruct((n_tokens, TOP_K), jnp.float32),
            jax.ShapeDtypeStruct((n_tokens, NUM_EXPERTS), jnp.float32),
            jax.ShapeDtypeStruct((n_tokens, TOP_C), jnp.int32),
        ],
        interpret=interpret,
    )(x, wt)
    idx8, w8, probs, idx16 = out
    return (idx8.astype(jnp.int64), w8, probs, idx16.astype(jnp.int64))


def kernel(x, W):
    return _run(x, W)


# speedup vs baseline: 1.7663x; 1.5914x over previous
import jax
import jax.numpy as jnp
from jax.experimental import pallas as pl

D_MODEL = 4096
NUM_EXPERTS = 64
TOP_K = 8
TOP_C = 16

_NEG_BIG = -3.0e38


def _gating_body(x_ref, wt_ref, idx8_ref, w8_ref, probs_ref, idx16_ref):
    logits = jnp.dot(x_ref[...], wt_ref[...],
                     preferred_element_type=jnp.float32)

    lt = logits.T

    iota = jax.lax.broadcasted_iota(jnp.int32, lt.shape, 0)
    vals = lt
    tv, ti = [], []
    for _ in range(TOP_C):
        m = jnp.max(vals, axis=0, keepdims=True)
        idx = jnp.min(jnp.where(vals == m, iota, NUM_EXPERTS),
                      axis=0, keepdims=True)
        tv.append(m)
        ti.append(idx)
        vals = jnp.where(iota == idx, _NEG_BIG, vals)

    e = jnp.exp(lt - tv[0])
    s = jnp.sum(e, axis=0, keepdims=True)
    probs_ref[...] = (e / (s + 1e-12)).T

    idx16_ref[...] = jnp.concatenate(ti, axis=0).T
    idx8_ref[...] = jnp.concatenate(ti[:TOP_K], axis=0).T

    topv = jnp.concatenate(tv[:TOP_K], axis=0)
    e8 = jnp.exp(topv - tv[0])
    w8_ref[...] = (e8 / (jnp.sum(e8, axis=0, keepdims=True) + 1e-12)).T


def _run(x, W, block_m=512, interpret=False):
    n_tokens = x.shape[0]
    wt = W.T
    grid = (n_tokens // block_m,)
    out = pl.pallas_call(
        _gating_body,
        grid=grid,
        in_specs=[
            pl.BlockSpec((block_m, D_MODEL), lambda i: (i, 0)),
            pl.BlockSpec((D_MODEL, NUM_EXPERTS), lambda i: (0, 0)),
        ],
        out_specs=[
            pl.BlockSpec((block_m, TOP_K), lambda i: (i, 0)),
            pl.BlockSpec((block_m, TOP_K), lambda i: (i, 0)),
            pl.BlockSpec((block_m, NUM_EXPERTS), lambda i: (i, 0)),
            pl.BlockSpec((block_m, TOP_C), lambda i: (i, 0)),
        ],
        out_shape=[
            jax.ShapeDtypeStruct((n_tokens, TOP_K), jnp.int32),
            jax.ShapeDtypeStruct((n_tokens, TOP_K), jnp.float32),
            jax.ShapeDtypeStruct((n_tokens, NUM_EXPERTS), jnp.float32),
            jax.ShapeDtypeStruct((n_tokens, TOP_C), jnp.int32),
        ],
        interpret=interpret,
    )(x, wt)
    idx8, w8, probs, idx16 = out
    return (idx8.astype(jnp.int64), w8, probs, idx16.astype(jnp.int64))


def kernel(x, W):
    return _run(x, W)


# speedup vs baseline: 1.7809x; 1.0083x over previous
import jax
import jax.numpy as jnp
from jax.experimental import pallas as pl
from jax.experimental.pallas import tpu as pltpu

D_MODEL = 4096
NUM_EXPERTS = 64
TOP_K = 8
TOP_C = 16

_NEG_BIG = -3.0e38


def _gating_body(x_ref, wt_ref, idx8_ref, w8_ref, probs_ref, idx16_ref):
    logits = jnp.dot(x_ref[...], wt_ref[...],
                     preferred_element_type=jnp.float32)

    lt = logits.T

    iota = jax.lax.broadcasted_iota(jnp.int32, lt.shape, 0)
    vals = lt
    tv, ti = [], []
    for _ in range(TOP_C):
        m = jnp.max(vals, axis=0, keepdims=True)
        idx = jnp.min(jnp.where(vals == m, iota, NUM_EXPERTS),
                      axis=0, keepdims=True)
        tv.append(m)
        ti.append(idx)
        vals = jnp.where(iota == idx, _NEG_BIG, vals)

    e = jnp.exp(lt - tv[0])
    s = jnp.sum(e, axis=0, keepdims=True)
    probs_ref[...] = (e / (s + 1e-12)).T

    idx16_ref[...] = jnp.concatenate(ti, axis=0).T
    idx8_ref[...] = jnp.concatenate(ti[:TOP_K], axis=0).T

    topv = jnp.concatenate(tv[:TOP_K], axis=0)
    e8 = jnp.exp(topv - tv[0])
    w8_ref[...] = (e8 / (jnp.sum(e8, axis=0, keepdims=True) + 1e-12)).T


def _run(x, W, block_m=512, interpret=False):
    n_tokens = x.shape[0]
    wt = W.T
    grid = (n_tokens // block_m,)
    out = pl.pallas_call(
        _gating_body,
        grid=grid,
        in_specs=[
            pl.BlockSpec((block_m, D_MODEL), lambda i: (i, 0)),
            pl.BlockSpec((D_MODEL, NUM_EXPERTS), lambda i: (0, 0)),
        ],
        out_specs=[
            pl.BlockSpec((block_m, TOP_K), lambda i: (i, 0)),
            pl.BlockSpec((block_m, TOP_K), lambda i: (i, 0)),
            pl.BlockSpec((block_m, NUM_EXPERTS), lambda i: (i, 0)),
            pl.BlockSpec((block_m, TOP_C), lambda i: (i, 0)),
        ],
        out_shape=[
            jax.ShapeDtypeStruct((n_tokens, TOP_K), jnp.int32),
            jax.ShapeDtypeStruct((n_tokens, TOP_K), jnp.float32),
            jax.ShapeDtypeStruct((n_tokens, NUM_EXPERTS), jnp.float32),
            jax.ShapeDtypeStruct((n_tokens, TOP_C), jnp.int32),
        ],
        compiler_params=pltpu.CompilerParams(
            dimension_semantics=(pltpu.PARALLEL,)),
        interpret=interpret,
    )(x, wt)
    idx8, w8, probs, idx16 = out
    return (idx8.astype(jnp.int64), w8, probs, idx16.astype(jnp.int64))


def kernel(x, W):
    return _run(x, W)


# speedup vs baseline: 1.8487x; 1.0380x over previous
import jax
import jax.numpy as jnp
from jax.experimental import pallas as pl
from jax.experimental.pallas import tpu as pltpu

D_MODEL = 4096
NUM_EXPERTS = 64
TOP_K = 8
TOP_C = 16

_NEG_BIG = -3.0e38


def _gating_body(x_ref, wt_ref, idx8_ref, w8_ref, probs_ref, idx16_ref):
    logits = jnp.dot(x_ref[...], wt_ref[...],
                     preferred_element_type=jnp.float32)

    lt = logits.T

    iota = jax.lax.broadcasted_iota(jnp.int32, lt.shape, 0)
    vals = lt
    tv, ti = [], []
    for _ in range(TOP_C):
        m = jnp.max(vals, axis=0, keepdims=True)
        idx = jnp.min(jnp.where(vals == m, iota, NUM_EXPERTS),
                      axis=0, keepdims=True)
        tv.append(m)
        ti.append(idx)
        vals = jnp.where(iota == idx, _NEG_BIG, vals)

    e = jnp.exp(lt - tv[0])
    s = jnp.sum(e, axis=0, keepdims=True)
    probs_ref[...] = (e / (s + 1e-12)).T

    idx16_ref[...] = jnp.concatenate(ti, axis=0).T
    idx8_ref[...] = jnp.concatenate(ti[:TOP_K], axis=0).T

    topv = jnp.concatenate(tv[:TOP_K], axis=0)
    e8 = jnp.exp(topv - tv[0])
    w8_ref[...] = (e8 / (jnp.sum(e8, axis=0, keepdims=True) + 1e-12)).T


def _run(x, W, block_m=1024, interpret=False):
    n_tokens = x.shape[0]
    wt = W.T
    grid = (n_tokens // block_m,)
    out = pl.pallas_call(
        _gating_body,
        grid=grid,
        in_specs=[
            pl.BlockSpec((block_m, D_MODEL), lambda i: (i, 0)),
            pl.BlockSpec((D_MODEL, NUM_EXPERTS), lambda i: (0, 0)),
        ],
        out_specs=[
            pl.BlockSpec((block_m, TOP_K), lambda i: (i, 0)),
            pl.BlockSpec((block_m, TOP_K), lambda i: (i, 0)),
            pl.BlockSpec((block_m, NUM_EXPERTS), lambda i: (i, 0)),
            pl.BlockSpec((block_m, TOP_C), lambda i: (i, 0)),
        ],
        out_shape=[
            jax.ShapeDtypeStruct((n_tokens, TOP_K), jnp.int32),
            jax.ShapeDtypeStruct((n_tokens, TOP_K), jnp.float32),
            jax.ShapeDtypeStruct((n_tokens, NUM_EXPERTS), jnp.float32),
            jax.ShapeDtypeStruct((n_tokens, TOP_C), jnp.int32),
        ],
        compiler_params=pltpu.CompilerParams(
            dimension_semantics=(pltpu.PARALLEL,)),
        interpret=interpret,
    )(x, wt)
    idx8, w8, probs, idx16 = out
    return (idx8.astype(jnp.int64), w8, probs, idx16.astype(jnp.int64))


def kernel(x, W):
    return _run(x, W)


# speedup vs baseline: 1.8528x; 1.0023x over previous
import jax
import jax.numpy as jnp
from jax.experimental import pallas as pl
from jax.experimental.pallas import tpu as pltpu

D_MODEL = 4096
D_HALF = D_MODEL // 2
NUM_EXPERTS = 64
TOP_K = 8
TOP_C = 16

_NEG_BIG = -3.0e38


def _gating_body(xa_ref, xb_ref, wt_ref, idx8_ref, w8_ref, probs_ref,
                 idx16_ref):
    logits = (
        jnp.dot(xa_ref[...], wt_ref[:D_HALF, :],
                preferred_element_type=jnp.float32)
        + jnp.dot(xb_ref[...], wt_ref[D_HALF:, :],
                  preferred_element_type=jnp.float32)
    )

    lt = logits.T

    iota = jax.lax.broadcasted_iota(jnp.int32, lt.shape, 0)
    vals = lt
    tv, ti = [], []
    for _ in range(TOP_C):
        m = jnp.max(vals, axis=0, keepdims=True)
        idx = jnp.min(jnp.where(vals == m, iota, NUM_EXPERTS),
                      axis=0, keepdims=True)
        tv.append(m)
        ti.append(idx)
        vals = jnp.where(iota == idx, _NEG_BIG, vals)

    e = jnp.exp(lt - tv[0])
    s = jnp.sum(e, axis=0, keepdims=True)
    probs_ref[...] = (e / (s + 1e-12)).T

    idx16_ref[...] = jnp.concatenate(ti, axis=0).T
    idx8_ref[...] = jnp.concatenate(ti[:TOP_K], axis=0).T

    topv = jnp.concatenate(tv[:TOP_K], axis=0)
    e8 = jnp.exp(topv - tv[0])
    w8_ref[...] = (e8 / (jnp.sum(e8, axis=0, keepdims=True) + 1e-12)).T


def _run(x, W, block_m=1024, interpret=False):
    n_tokens = x.shape[0]
    wt = W.T
    grid = (n_tokens // block_m,)
    out = pl.pallas_call(
        _gating_body,
        grid=grid,
        in_specs=[
            pl.BlockSpec((block_m, D_HALF), lambda i: (i, 0)),
            pl.BlockSpec((block_m, D_HALF), lambda i: (i, 1)),
            pl.BlockSpec((D_MODEL, NUM_EXPERTS), lambda i: (0, 0)),
        ],
        out_specs=[
            pl.BlockSpec((block_m, TOP_K), lambda i: (i, 0)),
            pl.BlockSpec((block_m, TOP_K), lambda i: (i, 0)),
            pl.BlockSpec((block_m, NUM_EXPERTS), lambda i: (i, 0)),
            pl.BlockSpec((block_m, TOP_C), lambda i: (i, 0)),
        ],
        out_shape=[
            jax.ShapeDtypeStruct((n_tokens, TOP_K), jnp.int32),
            jax.ShapeDtypeStruct((n_tokens, TOP_K), jnp.float32),
            jax.ShapeDtypeStruct((n_tokens, NUM_EXPERTS), jnp.float32),
            jax.ShapeDtypeStruct((n_tokens, TOP_C), jnp.int32),
        ],
        compiler_params=pltpu.CompilerParams(
            dimension_semantics=(pltpu.PARALLEL,)),
        interpret=interpret,
    )(x, x, wt)
    idx8, w8, probs, idx16 = out
    return (idx8.astype(jnp.int64), w8, probs, idx16.astype(jnp.int64))


def kernel(x, W):
    return _run(x, W)
